# baseline (speedup 1.0000x reference)
import functools

import jax
import jax.numpy as jnp
from jax import lax
from jax.experimental import pallas as pl
from jax.experimental.pallas import tpu as pltpu
from jax.experimental.pallas import tpu_sc as plsc

VOCAB = 100000
EMB = 64
CTX = 20
HID = 256
BATCH = 16384

CHUNK = 128
TILE = 16384
NT = (VOCAB + TILE - 1) // TILE


CROWS = 80
NSLOT = 4


def _sc_gather_sum(nc, per_w, idx_hbm, emb_hbm, out_hbm, idx_v, idx_f, rows_v,
                   acc_v, phase_v, *sems):
  nvec = EMB // 16
  rows_per_w = per_w // CTX
  nchunk = per_w // CROWS
  kper = CROWS // CTX
  nphase = CROWS // 16
  wid = lax.axis_index("s") * nc + lax.axis_index("c")

  pltpu.sync_copy(idx_hbm.at[pl.ds(wid * rows_per_w, rows_per_w), :], idx_v)

  lane = lax.iota(jnp.int32, 16)
  for q in range(nphase):
    coff = 16 * q + lane
    roff = jnp.zeros((16,), jnp.int32)
    for _ in range((16 * nphase - 1) // CTX):
      ge = coff >= CTX
      coff = jnp.where(ge, coff - CTX, coff)
      roff = jnp.where(ge, roff + 1, roff)
    phase_v[0, pl.ds(16 * q, 16)] = roff
    phase_v[1, pl.ds(16 * q, 16)] = coff

  def build(m, carry):
    for q in range(nphase):
      roff = phase_v[0, pl.ds(16 * q, 16)]
      coff = phase_v[1, pl.ds(16 * q, 16)]
      vals = plsc.load_gather(idx_v, [roff + kper * m, coff])
      idx_f[m, pl.ds(16 * q, 16)] = vals
    return carry

  lax.fori_loop(0, nchunk, build, 0)

  zero = jnp.zeros((16,), jnp.float32)
  for j in range(CTX):
    for v in range(nvec):
      acc_v[pl.ds(j * EMB + 16 * v, 16)] = zero

  def fire(c, s):
    pltpu.make_async_copy(emb_hbm.at[idx_f.at[c]], rows_v.at[s],
                          sems[s]).start()

  def drain(c, s):
    pltpu.make_async_copy(emb_hbm.at[idx_f.at[c]], rows_v.at[s],
                          sems[s]).wait()

  def accum(s):
    for jj in range(CTX):
      for v in range(nvec):
        sl = pl.ds(16 * v, 16)
        a = rows_v[s, jj, sl] + rows_v[s, CTX + jj, sl]
        for k in range(2, kper):
          a = a + rows_v[s, k * CTX + jj, sl]
        plsc.addupdate(acc_v.at[pl.ds(jj * EMB + 16 * v, 16)], a)

  for s in range(NSLOT):
    fire(s, s)

  def outer(g, carry):
    base = g * NSLOT
    for s in range(NSLOT):
      c = base + s
      drain(c, s)
      accum(s)

      @pl.when(c + NSLOT < nchunk)
      def _():
        fire(c + NSLOT, s)
    return carry

  lax.fori_loop(0, nchunk // NSLOT, outer, 0)
  pltpu.sync_copy(acc_v, out_hbm.at[wid])


def _sc_stage(inputs, emb, nw, nc, per_w):
  mesh = plsc.VectorSubcoreMesh(core_axis_name="c", subcore_axis_name="s")
  body = functools.partial(_sc_gather_sum, nc, per_w)
  return pl.kernel(
      body,
      out_type=jax.ShapeDtypeStruct((nw, CTX * EMB), jnp.float32),
      mesh=mesh,
      scratch_types=[
          pltpu.VMEM((per_w // CTX, CTX), jnp.int32),
          pltpu.VMEM((per_w // CROWS, CROWS), jnp.int32),
          pltpu.VMEM((NSLOT, CROWS, EMB), jnp.float32),
          pltpu.VMEM((CTX * EMB,), jnp.float32),
          pltpu.VMEM((2, CROWS), jnp.int32),
      ] + [pltpu.SemaphoreType.DMA] * NSLOT,
      compiler_params=pltpu.CompilerParams(use_tc_tiling_on_sc=False,
                                           needs_layout_passes=False),
  )(inputs, emb)


def _tc_body(nw, partials_ref, w1_ref, b1_ref, w2_ref, b2_ref, out_ref,
             logits_ref, h_ref, m_ref, s_ref):
  p = pl.program_id(0)
  t = pl.program_id(1)

  @pl.when(jnp.logical_and(p == 0, t == 0))
  def _init():
    sums = jnp.sum(partials_ref[...], axis=0, keepdims=True)
    embeds = sums * (1.0 / BATCH)
    hid = lax.dot_general(embeds, w1_ref[...], (((1,), (1,)), ((), ())),
                          preferred_element_type=jnp.float32)
    h_ref[...] = jnp.maximum(hid + b1_ref[...].reshape(1, HID), 0.0)
    m_ref[0] = -jnp.inf
    s_ref[0] = 0.0

  @pl.when(p == 0)
  def _phase0():
    logits = lax.dot_general(h_ref[...], w2_ref[...], (((1,), (1,)), ((), ())),
                             preferred_element_type=jnp.float32)
    logits = logits + b2_ref[...].reshape(1, TILE)
    col = t * TILE + lax.broadcasted_iota(jnp.int32, (1, TILE), 1)
    logits = jnp.where(col < VOCAB, logits, -jnp.inf)
    logits_ref[t] = logits
    m_old = m_ref[0]
    m_new = jnp.maximum(m_old, jnp.max(logits))
    s_ref[0] = (s_ref[0] * jnp.exp(m_old - m_new)
                + jnp.sum(jnp.exp(logits - m_new)))
    m_ref[0] = m_new

  @pl.when(p == 1)
  def _phase1():
    lse = m_ref[0] + jnp.log(s_ref[0])
    out_ref[...] = logits_ref[t] - lse


def _tc_stage(partials, w1, b1, w2, b2, nw):
  grid = (2, NT)
  return pl.pallas_call(
      functools.partial(_tc_body, nw),
      grid=grid,
      in_specs=[
          pl.BlockSpec((nw, CTX * EMB), lambda p, t: (0, 0)),
          pl.BlockSpec((HID, CTX * EMB), lambda p, t: (0, 0)),
          pl.BlockSpec((HID,), lambda p, t: (0,)),
          pl.BlockSpec((TILE, HID),
                       lambda p, t: (jnp.where(p == 0, t, NT - 1), 0)),
          pl.BlockSpec((TILE,), lambda p, t: (jnp.where(p == 0, t, NT - 1),)),
      ],
      out_specs=pl.BlockSpec((1, TILE), lambda p, t: (0, t)),
      out_shape=jax.ShapeDtypeStruct((1, VOCAB), jnp.float32),
      scratch_shapes=[
          pltpu.VMEM((NT, 1, TILE), jnp.float32),
          pltpu.VMEM((1, HID), jnp.float32),
          pltpu.SMEM((1,), jnp.float32),
          pltpu.SMEM((1,), jnp.float32),
      ],
  )(partials, w1, b1, w2, b2)


def kernel(inputs, emb, W1, b1, W2, b2):
  info = plsc.get_sparse_core_info()
  nw = info.num_cores * info.num_subcores
  per_w = BATCH * CTX // nw
  partials = _sc_stage(inputs, emb, nw, info.num_cores, per_w)
  return _tc_stage(partials, W1, b1, W2, b2, nw)


# speedup vs baseline: 1.7859x; 1.7859x over previous
import functools

import jax
import jax.numpy as jnp
from jax import lax
from jax.experimental import pallas as pl
from jax.experimental.pallas import tpu as pltpu
from jax.experimental.pallas import tpu_sc as plsc

VOCAB = 100000
EMB = 64
CTX = 20
HID = 256
BATCH = 16384

CHUNK = 128
NBUF = 3
TILE = 16384
NT = (VOCAB + TILE - 1) // TILE


def _sc_gather_sum(nc, per_w, idx_hbm, emb_hbm, out_hbm, idx_v, rows_v, acc_v,
                   *sems):
  nch = per_w // CHUNK
  nvec = EMB // 16
  wid = lax.axis_index("s") * nc + lax.axis_index("c")

  pltpu.sync_copy(idx_hbm.at[:, pl.ds(wid * per_w, per_w)], idx_v)

  def start_j(j):
    b = j % NBUF
    cps = [
        pltpu.make_async_copy(
            emb_hbm.at[idx_v.at[j, pl.ds(c * CHUNK, CHUNK)]],
            rows_v.at[b, c], sems[b]) for c in range(nch)
    ]
    for cp in cps:
      cp.start()
    return cps

  zero = jnp.zeros((16,), jnp.float32)
  pend = [start_j(0), start_j(1)]
  for j in range(CTX):
    if j + 2 < CTX:
      pend.append(start_j(j + 2))
    for cp in pend.pop(0):
      cp.wait()
    b = j % NBUF
    acc = (zero,) * nvec
    for c in range(nch):
      def row_body(r, a, _b=b, _c=c):
        return tuple(a[v] + rows_v[_b, _c, r, pl.ds(16 * v, 16)]
                     for v in range(nvec))
      acc = lax.fori_loop(0, CHUNK, row_body, acc)
    for v in range(nvec):
      acc_v[pl.ds(j * EMB + 16 * v, 16)] = acc[v]
  pltpu.sync_copy(acc_v, out_hbm.at[wid])


def _sc_stage(inputs_t, emb, nw, nc, per_w):
  mesh = plsc.VectorSubcoreMesh(core_axis_name="c", subcore_axis_name="s")
  body = functools.partial(_sc_gather_sum, nc, per_w)
  return pl.kernel(
      body,
      out_type=jax.ShapeDtypeStruct((nw, CTX * EMB), jnp.float32),
      mesh=mesh,
      scratch_types=[
          pltpu.VMEM((CTX, per_w), jnp.int32),
          pltpu.VMEM((NBUF, per_w // CHUNK, CHUNK, EMB), jnp.float32),
          pltpu.VMEM((CTX * EMB,), jnp.float32),
      ] + [pltpu.SemaphoreType.DMA] * NBUF,
      compiler_params=pltpu.CompilerParams(use_tc_tiling_on_sc=False),
  )(inputs_t, emb)


def _tc_body(nw, partials_ref, w1_ref, b1_ref, w2_ref, b2_ref, out_ref,
             logits_ref, h_ref, m_ref, s_ref):
  p = pl.program_id(0)
  t = pl.program_id(1)

  @pl.when(jnp.logical_and(p == 0, t == 0))
  def _init():
    sums = jnp.sum(partials_ref[...], axis=0, keepdims=True)
    embeds = sums * (1.0 / BATCH)
    hid = lax.dot_general(embeds, w1_ref[...], (((1,), (1,)), ((), ())),
                          preferred_element_type=jnp.float32)
    h_ref[...] = jnp.maximum(hid + b1_ref[...].reshape(1, HID), 0.0)
    m_ref[0] = -jnp.inf
    s_ref[0] = 0.0

  @pl.when(p == 0)
  def _phase0():
    logits = lax.dot_general(h_ref[...], w2_ref[...], (((1,), (1,)), ((), ())),
                             preferred_element_type=jnp.float32)
    logits = logits + b2_ref[...].reshape(1, TILE)
    col = t * TILE + lax.broadcasted_iota(jnp.int32, (1, TILE), 1)
    logits = jnp.where(col < VOCAB, logits, -jnp.inf)
    logits_ref[t] = logits
    m_old = m_ref[0]
    m_new = jnp.maximum(m_old, jnp.max(logits))
    s_ref[0] = (s_ref[0] * jnp.exp(m_old - m_new)
                + jnp.sum(jnp.exp(logits - m_new)))
    m_ref[0] = m_new

  @pl.when(p == 1)
  def _phase1():
    lse = m_ref[0] + jnp.log(s_ref[0])
    out_ref[...] = logits_ref[t] - lse


def _tc_stage(partials, w1, b1, w2, b2, nw):
  grid = (2, NT)
  return pl.pallas_call(
      functools.partial(_tc_body, nw),
      grid=grid,
      in_specs=[
          pl.BlockSpec((nw, CTX * EMB), lambda p, t: (0, 0)),
          pl.BlockSpec((HID, CTX * EMB), lambda p, t: (0, 0)),
          pl.BlockSpec((HID,), lambda p, t: (0,)),
          pl.BlockSpec((TILE, HID),
                       lambda p, t: (jnp.where(p == 0, t, NT - 1), 0)),
          pl.BlockSpec((TILE,), lambda p, t: (jnp.where(p == 0, t, NT - 1),)),
      ],
      out_specs=pl.BlockSpec((1, TILE), lambda p, t: (0, t)),
      out_shape=jax.ShapeDtypeStruct((1, VOCAB), jnp.float32),
      scratch_shapes=[
          pltpu.VMEM((NT, 1, TILE), jnp.float32),
          pltpu.VMEM((1, HID), jnp.float32),
          pltpu.SMEM((1,), jnp.float32),
          pltpu.SMEM((1,), jnp.float32),
      ],
  )(partials, w1, b1, w2, b2)


def kernel(inputs, emb, W1, b1, W2, b2):
  info = plsc.get_sparse_core_info()
  nw = info.num_cores * info.num_subcores
  per_w = BATCH // nw
  inputs_t = inputs.astype(jnp.int32).T
  partials = _sc_stage(inputs_t, emb, nw, info.num_cores, per_w)
  return _tc_stage(partials, W1, b1, W2, b2, nw)


# speedup vs baseline: 1.7860x; 1.0000x over previous
import functools

import jax
import jax.numpy as jnp
from jax import lax
from jax.experimental import pallas as pl
from jax.experimental.pallas import tpu as pltpu
from jax.experimental.pallas import tpu_sc as plsc

VOCAB = 100000
EMB = 64
CTX = 20
HID = 256
BATCH = 16384

CHUNK = 128
NBUF = 3
TILE = 16384
NT = (VOCAB + TILE - 1) // TILE


def _sc_gather_sum(nc, per_w, idx_hbm, emb_hbm, out_hbm, idx_v, rows_v, acc_v,
                   *sems):
  nch = per_w // CHUNK
  nvec = EMB // 16
  wid = lax.axis_index("s") * nc + lax.axis_index("c")

  pltpu.sync_copy(idx_hbm.at[:, pl.ds(wid * per_w, per_w)], idx_v)

  def start_j(j):
    b = j % NBUF
    cps = [
        pltpu.make_async_copy(
            emb_hbm.at[idx_v.at[j, pl.ds(c * CHUNK, CHUNK)]],
            rows_v.at[b, c], sems[b]) for c in range(nch)
    ]
    for cp in cps:
      cp.start()
    return cps

  zero = jnp.zeros((16,), jnp.float32)
  pend = [start_j(0), start_j(1)]
  for j in range(CTX):
    if j + 2 < CTX:
      pend.append(start_j(j + 2))
    for cp in pend.pop(0):
      cp.wait()
    b = j % NBUF
    acc = (zero,) * nvec
    for c in range(nch):
      def row_body(r, a, _b=b, _c=c):
        return tuple(a[v] + rows_v[_b, _c, r, pl.ds(16 * v, 16)]
                     for v in range(nvec))
      acc = lax.fori_loop(0, CHUNK, row_body, acc)
    for v in range(nvec):
      acc_v[pl.ds(j * EMB + 16 * v, 16)] = acc[v]
  pltpu.sync_copy(acc_v, out_hbm.at[wid])


def _sc_stage(inputs_t, emb, nw, nc, per_w):
  mesh = plsc.VectorSubcoreMesh(core_axis_name="c", subcore_axis_name="s")
  body = functools.partial(_sc_gather_sum, nc, per_w)
  return pl.kernel(
      body,
      out_type=jax.ShapeDtypeStruct((nw, CTX * EMB), jnp.float32),
      mesh=mesh,
      scratch_types=[
          pltpu.VMEM((CTX, per_w), jnp.int32),
          pltpu.VMEM((NBUF, per_w // CHUNK, CHUNK, EMB), jnp.float32),
          pltpu.VMEM((CTX * EMB,), jnp.float32),
      ] + [pltpu.SemaphoreType.DMA] * NBUF,
      compiler_params=pltpu.CompilerParams(use_tc_tiling_on_sc=False),
  )(inputs_t, emb)


def _tc_body(nw, partials_ref, w1_ref, b1_ref, w2_ref, b2_ref, out_ref,
             logits_ref, h_ref, m_ref, s_ref):
  p = pl.program_id(0)
  t = pl.program_id(1)

  @pl.when(jnp.logical_and(p == 0, t == 0))
  def _init():
    sums = jnp.sum(partials_ref[...], axis=0, keepdims=True)
    embeds = sums * (1.0 / BATCH)
    hid = lax.dot_general(embeds, w1_ref[...], (((1,), (1,)), ((), ())),
                          preferred_element_type=jnp.float32)
    h_ref[...] = jnp.maximum(hid + b1_ref[...].reshape(1, HID), 0.0)
    m_ref[0] = -jnp.inf
    s_ref[0] = 0.0

  @pl.when(p == 0)
  def _phase0():
    logits = lax.dot_general(h_ref[...], w2_ref[...], (((1,), (1,)), ((), ())),
                             preferred_element_type=jnp.float32)
    logits = logits + b2_ref[...].reshape(1, TILE)
    col = t * TILE + lax.broadcasted_iota(jnp.int32, (1, TILE), 1)
    logits = jnp.where(col < VOCAB, logits, -jnp.inf)
    logits_ref[t] = logits
    m_old = m_ref[0]
    m_new = jnp.maximum(m_old, jnp.max(logits))
    s_ref[0] = (s_ref[0] * jnp.exp(m_old - m_new)
                + jnp.sum(jnp.exp(logits - m_new)))
    m_ref[0] = m_new

  @pl.when(p == 1)
  def _phase1():
    lse = m_ref[0] + jnp.log(s_ref[0])
    out_ref[...] = logits_ref[t] - lse


def _tc_stage(partials, w1, b1, w2, b2, nw):
  grid = (2, NT)
  return pl.pallas_call(
      functools.partial(_tc_body, nw),
      grid=grid,
      in_specs=[
          pl.BlockSpec((nw, CTX * EMB), lambda p, t: (0, 0)),
          pl.BlockSpec((HID, CTX * EMB), lambda p, t: (0, 0)),
          pl.BlockSpec((HID,), lambda p, t: (0,)),
          pl.BlockSpec((TILE, HID),
                       lambda p, t: (jnp.where(p == 0, t, NT - 1), 0)),
          pl.BlockSpec((TILE,), lambda p, t: (jnp.where(p == 0, t, NT - 1),)),
      ],
      out_specs=pl.BlockSpec((1, TILE), lambda p, t: (0, t)),
      out_shape=jax.ShapeDtypeStruct((1, VOCAB), jnp.float32),
      scratch_shapes=[
          pltpu.VMEM((NT, 1, TILE), jnp.float32),
          pltpu.VMEM((1, HID), jnp.float32),
          pltpu.SMEM((1,), jnp.float32),
          pltpu.SMEM((1,), jnp.float32),
      ],
  )(partials, w1, b1, w2, b2)


def kernel(inputs, emb, W1, b1, W2, b2):
  info = plsc.get_sparse_core_info()
  nw = info.num_cores * info.num_subcores
  per_w = BATCH // nw
  inputs_t = inputs.astype(jnp.int32).T
  partials = _sc_stage(inputs_t, emb + 0.0, nw, info.num_cores, per_w)
  return _tc_stage(partials, W1, b1, W2, b2, nw)
